# baseline (speedup 1.0000x reference)
import functools

import jax
import jax.numpy as jnp
from jax import lax
from jax.experimental import pallas as pl
from jax.experimental.pallas import tpu as pltpu
from jax.experimental.pallas import tpu_sc as plsc

_NC = 2
_NS = 16
_NW = _NC * _NS
_LANES = 16
_HIMASK = -65536
_NBUF = 3


def _pack_rows(y, d):
    u = lax.bitcast_convert_type(y, jnp.int32)
    r = u + 0x7FFF + jnp.bitwise_and(jnp.right_shift(u, 16), 1)
    lo = jnp.bitwise_and(jnp.right_shift(r[:, : d // 2], 16), 0xFFFF)
    hi = jnp.bitwise_and(r[:, d // 2:], _HIMASK)
    return jnp.bitwise_or(lo, hi)


def _mlp_body(x_ref, w_ref, b_ref, o_ref):
    d = w_ref.shape[1]
    y = jnp.dot(x_ref[...], w_ref[...], preferred_element_type=jnp.float32)
    y = jnp.maximum(y + b_ref[...], 0.0)
    y3 = y.reshape(y.shape[0] // 2, 2, d)
    o_ref[:, : d // 2] = _pack_rows(y3[:, 0, :], d)
    o_ref[:, d // 2:] = _pack_rows(y3[:, 1, :], d)


def _mlp_packed(feat, W, b, block_rows=2000):
    n, d_in = feat.shape
    d_hid = W.shape[1]
    assert n % block_rows == 0
    grid = n // block_rows
    return pl.pallas_call(
        _mlp_body,
        grid=(grid,),
        in_specs=[
            pl.BlockSpec((block_rows, d_in), lambda i: (i, 0)),
            pl.BlockSpec((d_in, d_hid), lambda i: (0, 0)),
            pl.BlockSpec((1, d_hid), lambda i: (0, 0)),
        ],
        out_specs=pl.BlockSpec((block_rows // 2, d_hid), lambda i: (i, 0)),
        out_shape=jax.ShapeDtypeStruct((n // 2, d_hid), jnp.int32),
    )(feat, W, b.reshape(1, d_hid))


@functools.partial(jax.jit,
                   static_argnames=("s", "d", "n_batch", "chunk_rows",
                                    "n_chunks"))
def _sc_pool(h_packed, idx_2d, s, d, n_batch, chunk_rows, n_chunks):
    half_rows = chunk_rows // 2
    iph = half_rows * s
    rpw = chunk_rows * n_chunks
    dw = d // 2
    ngroups = dw // _LANES
    last_start = n_batch - chunk_rows
    stage_floor = n_batch - rpw

    assert n_chunks % _NBUF == 0 and n_chunks >= 2 * _NBUF
    assert iph <= 128 and chunk_rows % 8 == 0
    assert last_start % 8 == 0 and stage_floor % 8 == 0

    @functools.partial(
        pl.kernel,
        out_type=jax.ShapeDtypeStruct((n_batch, d), jnp.float32),
        mesh=plsc.VectorSubcoreMesh(
            core_axis_name="c", subcore_axis_name="s",
            num_cores=_NC, num_subcores=_NS),
        compiler_params=pltpu.CompilerParams(use_tc_tiling_on_sc=False, needs_layout_passes=False),
        scratch_types=[
            pltpu.VMEM((rpw * s,), jnp.int32),
            pltpu.VMEM((_NBUF, 2 * iph, dw), jnp.int32),
            pltpu.VMEM((_NBUF, chunk_rows, d), jnp.float32),
        ] + [pltpu.SemaphoreType.DMA] * (2 * _NBUF),
    )
    def _sc(h_hbm, idx_hbm, out_hbm, idx_v, gbufs, obufs, *sems):
        gsems = sems[:_NBUF]
        osems = sems[_NBUF:]
        wid = lax.axis_index("s") * _NC + lax.axis_index("c")
        row_base = wid * rpw
        stage_base = jnp.minimum(row_base, stage_floor)
        stage_off = pl.multiple_of(stage_base * s, 8)
        pltpu.sync_copy(idx_hbm.at[pl.ds(stage_off, rpw * s)], idx_v)

        def chunk_start(ch):
            start = jnp.minimum(row_base + ch * chunk_rows, last_start)
            return pl.multiple_of(start, 8)

        def out_slice(ch):
            return out_hbm.at[pl.ds(chunk_start(ch), chunk_rows)]

        def idx_slice(ch, half):
            off = (chunk_start(ch) - stage_base) * s + half * iph
            return idx_v.at[pl.ds(pl.multiple_of(off, 8), iph)]

        def start_gather(ch, buf):
            pltpu.async_copy(h_hbm.at[idx_slice(ch, 0)],
                             gbufs.at[buf].at[pl.ds(0, iph)], gsems[buf])
            pltpu.async_copy(h_hbm.at[idx_slice(ch, 1)],
                             gbufs.at[buf].at[pl.ds(iph, iph)], gsems[buf])

        def wait_gather(ch, buf):
            pltpu.make_async_copy(h_hbm.at[idx_slice(ch, 0)],
                                  gbufs.at[buf].at[pl.ds(0, iph)],
                                  gsems[buf]).wait()
            pltpu.make_async_copy(h_hbm.at[idx_slice(ch, 1)],
                                  gbufs.at[buf].at[pl.ds(iph, iph)],
                                  gsems[buf]).wait()

        for p in range(_NBUF - 1):
            start_gather(p, p)

        def ring_body(i, carry):
            for p in range(_NBUF):
                ch = _NBUF * i + p

                @pl.when(ch + _NBUF - 1 < n_chunks)
                def _():
                    start_gather(ch + _NBUF - 1, (p + _NBUF - 1) % _NBUF)

                wait_gather(ch, p)

                @pl.when(ch >= _NBUF)
                def _():
                    pltpu.make_async_copy(
                        obufs.at[p], out_slice(ch - _NBUF), osems[p]).wait()

                gbuf = gbufs.at[p]
                obuf = obufs.at[p]

                def row_body(c2, carry2):
                    b0 = c2 * 2 * s
                    b1 = b0 + s
                    for g in range(ngroups):
                        sl = pl.ds(g * _LANES, _LANES)
                        w0a = gbuf[b0, sl]
                        w0b = gbuf[b1, sl]
                        hia, hib = w0a, w0b
                        loa = jnp.left_shift(w0a, 16)
                        lob = jnp.left_shift(w0b, 16)
                        for j in range(1, s):
                            wa = gbuf[b0 + j, sl]
                            wb = gbuf[b1 + j, sl]
                            hia = jnp.maximum(hia, wa)
                            hib = jnp.maximum(hib, wb)
                            loa = jnp.maximum(loa, jnp.left_shift(wa, 16))
                            lob = jnp.maximum(lob, jnp.left_shift(wb, 16))
                        obuf[2 * c2, pl.ds(g * _LANES, _LANES)] = (
                            plsc.bitcast(loa, jnp.float32))
                        obuf[2 * c2 + 1, pl.ds(g * _LANES, _LANES)] = (
                            plsc.bitcast(lob, jnp.float32))
                        obuf[2 * c2, pl.ds(dw + g * _LANES, _LANES)] = (
                            plsc.bitcast(jnp.bitwise_and(hia, _HIMASK),
                                         jnp.float32))
                        obuf[2 * c2 + 1, pl.ds(dw + g * _LANES, _LANES)] = (
                            plsc.bitcast(jnp.bitwise_and(hib, _HIMASK),
                                         jnp.float32))
                    return carry2

                lax.fori_loop(0, chunk_rows // 2, row_body, 0)

                pltpu.async_copy(obufs.at[p], out_slice(ch), osems[p])
            return carry

        lax.fori_loop(0, n_chunks // _NBUF, ring_body, 0)

        for p in range(_NBUF):
            pltpu.make_async_copy(
                obufs.at[p], out_slice(n_chunks - _NBUF + p), osems[p]).wait()

    return _sc(h_packed, idx_2d)


def kernel(neigh_idx, feat_table, W, b):
    n_batch, s = neigh_idx.shape
    n_nodes, d_in = feat_table.shape
    d_hid = W.shape[1]

    h2 = _mlp_packed(feat_table, W, b)
    h_packed = h2.reshape(n_nodes, d_hid // 2)

    chunk_rows = 24
    per_chunk = _NW * chunk_rows
    n_chunks = -(-n_batch // per_chunk)
    n_chunks += (-n_chunks) % _NBUF

    idx_flat = neigh_idx.astype(jnp.int32).reshape(-1)
    return _sc_pool(h_packed, idx_flat, s, d_hid, n_batch, chunk_rows,
                    n_chunks)


# speedup vs baseline: 1.0882x; 1.0882x over previous
import functools

import jax
import jax.numpy as jnp
from jax import lax
from jax.experimental import pallas as pl
from jax.experimental.pallas import tpu as pltpu
from jax.experimental.pallas import tpu_sc as plsc

_NC = 2
_NS = 16
_NW = _NC * _NS
_LANES = 16
_HIMASK = -65536
_NBUF = 3


def _pack_rows(y, d):
    u = lax.bitcast_convert_type(y, jnp.int32)
    r = u + 0x7FFF + jnp.bitwise_and(jnp.right_shift(u, 16), 1)
    lo = jnp.bitwise_and(jnp.right_shift(r[:, : d // 2], 16), 0xFFFF)
    hi = jnp.bitwise_and(r[:, d // 2:], _HIMASK)
    return jnp.bitwise_or(lo, hi)


def _mlp_body(x1_ref, x2_ref, w_ref, b_ref, o_ref):
    d = w_ref.shape[1]
    y1 = jnp.dot(x1_ref[...], w_ref[...], preferred_element_type=jnp.float32)
    y2 = jnp.dot(x2_ref[...], w_ref[...], preferred_element_type=jnp.float32)
    y1 = jnp.maximum(y1 + b_ref[...], 0.0)
    y2 = jnp.maximum(y2 + b_ref[...], 0.0)
    o_ref[:, : d // 2] = _pack_rows(y1, d)
    o_ref[:, d // 2:] = _pack_rows(y2, d)


def _mlp_packed(feat, W, b, block_rows=1000):
    n, d_in = feat.shape
    d_hid = W.shape[1]
    half = n // 2
    assert half % block_rows == 0
    grid = half // block_rows
    return pl.pallas_call(
        _mlp_body,
        grid=(grid,),
        in_specs=[
            pl.BlockSpec((block_rows, d_in), lambda i: (i, 0)),
            pl.BlockSpec((block_rows, d_in),
                         lambda i, g=grid: (i + g, 0)),
            pl.BlockSpec((d_in, d_hid), lambda i: (0, 0)),
            pl.BlockSpec((1, d_hid), lambda i: (0, 0)),
        ],
        out_specs=pl.BlockSpec((block_rows, d_hid), lambda i: (i, 0)),
        out_shape=jax.ShapeDtypeStruct((half, d_hid), jnp.int32),
    )(feat, feat, W, b.reshape(1, d_hid))


@functools.partial(jax.jit,
                   static_argnames=("s", "d", "n_batch", "n_nodes",
                                    "chunk_rows", "n_chunks"))
def _sc_pool(h_packed, idx_flat, s, d, n_batch, n_nodes, chunk_rows,
             n_chunks):
    half_nodes = n_nodes // 2
    half_rows = chunk_rows // 2
    iph = half_rows * s
    rpw = chunk_rows * n_chunks
    dw = d // 2
    ngroups = dw // _LANES
    last_start = n_batch - chunk_rows
    stage_floor = n_batch - rpw

    assert n_chunks % _NBUF == 0 and n_chunks >= 2 * _NBUF
    assert iph <= 128 and chunk_rows % 8 == 0
    assert last_start % 8 == 0 and stage_floor % 8 == 0

    @functools.partial(
        pl.kernel,
        out_type=jax.ShapeDtypeStruct((n_batch, d), jnp.float32),
        mesh=plsc.VectorSubcoreMesh(
            core_axis_name="c", subcore_axis_name="s",
            num_cores=_NC, num_subcores=_NS),
        compiler_params=pltpu.CompilerParams(use_tc_tiling_on_sc=False, needs_layout_passes=False),
        scratch_types=[
            pltpu.VMEM((rpw * s,), jnp.int32),
            pltpu.VMEM((_NBUF, 2 * iph, dw), jnp.int32),
            pltpu.VMEM((_NBUF, chunk_rows, d), jnp.float32),
        ] + [pltpu.SemaphoreType.DMA] * (2 * _NBUF),
    )
    def _sc(h_hbm, idx_hbm, out_hbm, idx_v, gbufs, obufs, *sems):
        gsems = sems[:_NBUF]
        osems = sems[_NBUF:]
        wid = lax.axis_index("s") * _NC + lax.axis_index("c")
        row_base = wid * rpw
        stage_base = jnp.minimum(row_base, stage_floor)
        stage_off = pl.multiple_of(stage_base * s, 8)
        pltpu.sync_copy(idx_hbm.at[pl.ds(stage_off, rpw * s)], idx_v)

        def xform_body(t, carry0):
            slt = pl.ds(t * _LANES, _LANES)
            v = idx_v[slt]
            wrap = jnp.where(v >= half_nodes, n_nodes - 1, 0)
            idx_v[slt] = 2 * v - wrap
            return carry0

        lax.fori_loop(0, (rpw * s) // _LANES, xform_body, 0)

        def chunk_start(ch):
            start = jnp.minimum(row_base + ch * chunk_rows, last_start)
            return pl.multiple_of(start, 8)

        def out_slice(ch):
            return out_hbm.at[pl.ds(chunk_start(ch), chunk_rows)]

        def idx_slice(ch, half):
            off = (chunk_start(ch) - stage_base) * s + half * iph
            return idx_v.at[pl.ds(pl.multiple_of(off, 8), iph)]

        def start_gather(ch, buf):
            pltpu.async_copy(h_hbm.at[idx_slice(ch, 0)],
                             gbufs.at[buf].at[pl.ds(0, iph)], gsems[buf])
            pltpu.async_copy(h_hbm.at[idx_slice(ch, 1)],
                             gbufs.at[buf].at[pl.ds(iph, iph)], gsems[buf])

        def wait_gather(ch, buf):
            pltpu.make_async_copy(h_hbm.at[idx_slice(ch, 0)],
                                  gbufs.at[buf].at[pl.ds(0, iph)],
                                  gsems[buf]).wait()
            pltpu.make_async_copy(h_hbm.at[idx_slice(ch, 1)],
                                  gbufs.at[buf].at[pl.ds(iph, iph)],
                                  gsems[buf]).wait()

        for p in range(_NBUF - 1):
            start_gather(p, p)

        def ring_body(i, carry):
            for p in range(_NBUF):
                ch = _NBUF * i + p

                @pl.when(ch + _NBUF - 1 < n_chunks)
                def _():
                    start_gather(ch + _NBUF - 1, (p + _NBUF - 1) % _NBUF)

                wait_gather(ch, p)

                @pl.when(ch >= _NBUF)
                def _():
                    pltpu.make_async_copy(
                        obufs.at[p], out_slice(ch - _NBUF), osems[p]).wait()

                gbuf = gbufs.at[p]
                obuf = obufs.at[p]

                def row_body(c2, carry2):
                    b0 = c2 * 2 * s
                    b1 = b0 + s
                    for g in range(ngroups):
                        sl = pl.ds(g * _LANES, _LANES)
                        w0a = gbuf[b0, sl]
                        w0b = gbuf[b1, sl]
                        hia, hib = w0a, w0b
                        loa = jnp.left_shift(w0a, 16)
                        lob = jnp.left_shift(w0b, 16)
                        for j in range(1, s):
                            wa = gbuf[b0 + j, sl]
                            wb = gbuf[b1 + j, sl]
                            hia = jnp.maximum(hia, wa)
                            hib = jnp.maximum(hib, wb)
                            loa = jnp.maximum(loa, jnp.left_shift(wa, 16))
                            lob = jnp.maximum(lob, jnp.left_shift(wb, 16))
                        obuf[2 * c2, pl.ds(g * _LANES, _LANES)] = (
                            plsc.bitcast(loa, jnp.float32))
                        obuf[2 * c2 + 1, pl.ds(g * _LANES, _LANES)] = (
                            plsc.bitcast(lob, jnp.float32))
                        obuf[2 * c2, pl.ds(dw + g * _LANES, _LANES)] = (
                            plsc.bitcast(jnp.bitwise_and(hia, _HIMASK),
                                         jnp.float32))
                        obuf[2 * c2 + 1, pl.ds(dw + g * _LANES, _LANES)] = (
                            plsc.bitcast(jnp.bitwise_and(hib, _HIMASK),
                                         jnp.float32))
                    return carry2

                lax.fori_loop(0, chunk_rows // 2, row_body, 0)

                pltpu.async_copy(obufs.at[p], out_slice(ch), osems[p])
            return carry

        lax.fori_loop(0, n_chunks // _NBUF, ring_body, 0)

        for p in range(_NBUF):
            pltpu.make_async_copy(
                obufs.at[p], out_slice(n_chunks - _NBUF + p), osems[p]).wait()

    return _sc(h_packed, idx_flat)


def kernel(neigh_idx, feat_table, W, b):
    n_batch, s = neigh_idx.shape
    n_nodes, d_in = feat_table.shape
    d_hid = W.shape[1]

    h2 = _mlp_packed(feat_table, W, b)
    h_packed = h2.reshape(n_nodes, d_hid // 2)

    chunk_rows = 24
    per_chunk = _NW * chunk_rows
    n_chunks = -(-n_batch // per_chunk)
    n_chunks += (-n_chunks) % _NBUF

    idx_flat = neigh_idx.astype(jnp.int32).reshape(-1)
    return _sc_pool(h_packed, idx_flat, s, d_hid, n_batch, n_nodes,
                    chunk_rows, n_chunks)
